# baseline (speedup 1.0000x reference)
import jax
import jax.numpy as jnp
from jax import lax
from jax.experimental import pallas as pl
from jax.experimental.pallas import tpu as pltpu
from jax.experimental.pallas import tpu_sc as plsc

D = 64
B = 4096
NNEG = 50
KNBR = 10
NITEM = 100000
NUSER = 100000
W1 = 1e-6
W2 = 1.0
W3 = 1e-6
W4 = 1.0
NEG_WEIGHT = 10.0
GAMMA_REG = 1e-4
LAMBDA_ = 1e-3

NC = 2
NS = 16
L = 16
NW = NC * NS
BPW = B // NW
NV = BPW // L

_SC_PARAMS = pltpu.CompilerParams(use_tc_tiling_on_sc=False)


def _sc_mesh():
    return plsc.VectorSubcoreMesh(
        core_axis_name="c", subcore_axis_name="s",
        num_cores=NC, num_subcores=NS)


def _gather_u_rows(uflatT_h, u_idx, idxT, uT_rows, u_rows, sem, lane):
    def mk_idx(r, carry):
        d = r
        for v in range(NV):
            sl = pl.ds(v * L, L)
            idxT[r, sl] = u_idx[sl] + d * NUSER
        return carry
    lax.fori_loop(0, D, mk_idx, 0)

    def fire(d, carry):
        pltpu.async_copy(uflatT_h.at[idxT.at[d]], uT_rows.at[d], sem)
        return carry
    lax.fori_loop(0, D, fire, 0)

    def drain(d, carry):
        pltpu.make_async_copy(uflatT_h.at[idxT.at[0]], uT_rows.at[0],
                              sem).wait()
        return carry
    lax.fori_loop(0, D, drain, 0)

    masks = [((lane & s) == 0) for s in (8, 4, 2, 1)]

    def tr_block(bj, carry):
        col0 = bj * L
        for bi in range(D // L):
            v = [uT_rows[bi * L + r, pl.ds(col0, L)] for r in range(L)]
            for si, s in enumerate((8, 4, 2, 1)):
                m = masks[si]
                for r in range(L):
                    if r & s == 0:
                        a, b = v[r], v[r | s]
                        v[r] = jnp.where(m, a, b[lane ^ s])
                        v[r | s] = jnp.where(m, a[lane ^ s], b)
            for r in range(L):
                u_rows[col0 + r, pl.ds(bi * L, L)] = v[r]
        return carry
    lax.fori_loop(0, NV, tr_block, 0)


def _chunk_scores(u_rows, rows_ref, score_ref, sbase, lane, flat0, q, nsel):
    NK = D // L

    def dot_with(uslot, j):
        acc = None
        for k in range(NK):
            t = uslot(k) * rows_ref[j, pl.ds(k * L, L)]
            acc = t if acc is None else acc + t
        for sh in (8, 4, 2, 1):
            acc = acc + acc[lane ^ sh]
        return acc

    if q == 1:
        def body(j0, carry):
            vec = jnp.zeros((L,), jnp.float32)
            for jj in range(L):
                j = j0 * L + jj
                vec = jnp.where(
                    lane == jj,
                    dot_with(lambda k: u_rows[j, pl.ds(k * L, L)], j), vec)
            score_ref[pl.ds(sbase + j0 * L, L)] = vec
            return carry
        lax.fori_loop(0, NV, body, 0)
        return

    def body(j0, carry):
        g0 = flat0 + j0 * L
        b0 = g0 // q
        ubufs = []
        for t in range(nsel):
            bt = jnp.minimum(b0 + t, BPW - 1)
            ubufs.append([u_rows[bt, pl.ds(k * L, L)] for k in range(4)])
        vec = jnp.zeros((L,), jnp.float32)
        r0 = g0 - b0 * q
        for jj in range(L):
            s = (r0 + jj) // q

            def uslot(k, _s=s):
                uk = ubufs[0][k]
                for t in range(1, nsel):
                    uk = jnp.where(_s == t, ubufs[t][k], uk)
                return uk
            vec = jnp.where(lane == jj, dot_with(uslot, j0 * L + jj), vec)
        score_ref[pl.ds(sbase + j0 * L, L)] = vec
        return carry
    lax.fori_loop(0, NV, body, 0)


def _sc_main_body(users_h, pos_h, negf_h, neg0_h, uflatT_h, itab_h,
                  bu_h, bi_h,
                  ps_o, n0_o, bu_o, bip_o, negs_o, bin_o, urows_o,
                  u_idx, p_idx, n0_idx, negf_all, idxT, uT_rows,
                  u_rows, a_rows, b_rows, r0_rows, r1_rows, r2_rows,
                  r3_rows, bu_v, bip_v, scores_all, bin_all, sc_v,
                  semG, semA, semB, semBU, semBIP, semR0, semR1, semR2,
                  semR3, semF):
    wid = lax.axis_index("s") * NC + lax.axis_index("c")
    base = wid * BPW
    fbase_n = base * NNEG
    lane = lax.iota(jnp.int32, L)

    pltpu.sync_copy(users_h.at[pl.ds(base, BPW)], u_idx)
    pltpu.sync_copy(pos_h.at[pl.ds(base, BPW)], p_idx)
    pltpu.sync_copy(neg0_h.at[pl.ds(base, BPW)], n0_idx)
    cpA = pltpu.async_copy(itab_h.at[p_idx], a_rows, semA)
    cpB = pltpu.async_copy(itab_h.at[n0_idx], b_rows, semB)
    cpBU = pltpu.async_copy(bu_h.at[u_idx], bu_v, semBU)
    cpBIP = pltpu.async_copy(bi_h.at[p_idx], bip_v, semBIP)

    pltpu.sync_copy(negf_h.at[pl.ds(fbase_n, BPW * NNEG)], negf_all)

    def idx_of(c):
        return negf_all.at[pl.ds(c * BPW, BPW)]

    rbufs = (r0_rows, r1_rows, r2_rows, r3_rows)
    rsems = (semR0, semR1, semR2, semR3)
    for i in range(4):
        pltpu.async_copy(itab_h.at[idx_of(i)], rbufs[i], rsems[i])

    _gather_u_rows(uflatT_h, u_idx, idxT, uT_rows, u_rows, semG, lane)
    pltpu.sync_copy(u_rows, urows_o.at[pl.ds(base, BPW), :])

    cpA.wait()
    _chunk_scores(u_rows, a_rows, sc_v, 0, lane, 0, 1, 1)
    pltpu.sync_copy(sc_v, ps_o.at[pl.ds(base, BPW)])
    cpB.wait()
    _chunk_scores(u_rows, b_rows, sc_v, 0, lane, 0, 1, 1)
    pltpu.sync_copy(sc_v, n0_o.at[pl.ds(base, BPW)])
    cpBU.wait()
    pltpu.sync_copy(bu_v, bu_o.at[pl.ds(base, BPW)])
    cpBIP.wait()
    pltpu.sync_copy(bip_v, bip_o.at[pl.ds(base, BPW)])

    def consume(c, rows, semr, prefetch):
        off = c * BPW
        pltpu.make_async_copy(itab_h.at[idx_of(c)], rows, semr).wait()
        pltpu.async_copy(bi_h.at[idx_of(c)],
                         bin_all.at[pl.ds(off, BPW)], semF)
        _chunk_scores(u_rows, rows, scores_all, off, lane, off, NNEG, 2)
        if prefetch:
            @pl.when(c + 4 < NNEG)
            def _():
                pltpu.async_copy(itab_h.at[idx_of(c + 4)], rows, semr)

    def outer(cc, carry):
        e = cc * 4
        for i in range(4):
            consume(e + i, rbufs[i], rsems[i], True)
        return carry

    lax.fori_loop(0, NNEG // 4, outer, 0)
    consume(NNEG - 2, r0_rows, semR0, False)
    consume(NNEG - 1, r1_rows, semR1, False)

    pltpu.sync_copy(scores_all, negs_o.at[pl.ds(fbase_n, BPW * NNEG)])

    def drainF(c, carry):
        pltpu.make_async_copy(bi_h.at[idx_of(0)],
                              bin_all.at[pl.ds(0, BPW)], semF).wait()
        return carry
    lax.fori_loop(0, NNEG, drainF, 0)
    pltpu.sync_copy(bin_all, bin_o.at[pl.ds(fbase_n, BPW * NNEG)])


def _sc_main_call(users, pos, negf, neg0, uflatT, itab, bu, bi):
    f32 = jnp.float32
    i32 = jnp.int32
    out_type = [
        jax.ShapeDtypeStruct((B,), f32),
        jax.ShapeDtypeStruct((B,), f32),
        jax.ShapeDtypeStruct((B,), f32),
        jax.ShapeDtypeStruct((B,), f32),
        jax.ShapeDtypeStruct((B * NNEG,), f32),
        jax.ShapeDtypeStruct((B * NNEG,), f32),
        jax.ShapeDtypeStruct((B, D), f32),
    ]
    scratch_types = [
        pltpu.VMEM((BPW,), i32),
        pltpu.VMEM((BPW,), i32),
        pltpu.VMEM((BPW,), i32),
        pltpu.VMEM((BPW * NNEG,), i32),
        pltpu.VMEM((D, BPW), i32),
        pltpu.VMEM((D, BPW), f32),
        pltpu.VMEM((BPW, D), f32),
        pltpu.VMEM((BPW, D), f32),
        pltpu.VMEM((BPW, D), f32),
        pltpu.VMEM((BPW, D), f32),
        pltpu.VMEM((BPW, D), f32),
        pltpu.VMEM((BPW, D), f32),
        pltpu.VMEM((BPW, D), f32),
        pltpu.VMEM((BPW,), f32),
        pltpu.VMEM((BPW,), f32),
        pltpu.VMEM((BPW * NNEG,), f32),
        pltpu.VMEM((BPW * NNEG,), f32),
        pltpu.VMEM((BPW,), f32),
    ] + [pltpu.SemaphoreType.DMA] * 10
    fn = pl.kernel(_sc_main_body, out_type=out_type, mesh=_sc_mesh(),
                   scratch_types=scratch_types, compiler_params=_SC_PARAMS)
    return fn(users, pos, negf, neg0, uflatT, itab, bu, bi)


def _sc_nbr_body(urows_h, nbrpos_h, itab_h, nbrf_h, simf_h,
                 inner_o, sim_o,
                 u_rows, np_all, ids_all,
                 r0_rows, r1_rows, r2_rows, r3_rows, scores_all, sim_all,
                 semU, semI, semR0, semR1, semR2, semR3, semS):
    wid = lax.axis_index("s") * NC + lax.axis_index("c")
    base = wid * BPW
    fbase = base * KNBR
    lane = lax.iota(jnp.int32, L)

    cpU = pltpu.async_copy(urows_h.at[pl.ds(base, BPW), :], u_rows, semU)
    pltpu.sync_copy(nbrpos_h.at[pl.ds(fbase, BPW * KNBR)], np_all)

    def pidx_of(c):
        return np_all.at[pl.ds(c * BPW, BPW)]

    def ids_of(c):
        return ids_all.at[pl.ds(c * BPW, BPW)]

    for c in range(KNBR):
        pltpu.async_copy(nbrf_h.at[pidx_of(c)], ids_of(c), semI)
        pltpu.async_copy(simf_h.at[pidx_of(c)],
                         sim_all.at[pl.ds(c * BPW, BPW)], semS)

    def drainI(c, carry):
        pltpu.make_async_copy(nbrf_h.at[pidx_of(0)], ids_of(0),
                              semI).wait()
        return carry
    lax.fori_loop(0, KNBR, drainI, 0)

    rbufs = (r0_rows, r1_rows, r2_rows, r3_rows)
    rsems = (semR0, semR1, semR2, semR3)
    for i in range(4):
        pltpu.async_copy(itab_h.at[ids_of(i)], rbufs[i], rsems[i])
    cpU.wait()

    def step(c, rows, semr):
        off = c * BPW
        pltpu.make_async_copy(itab_h.at[ids_of(c)], rows, semr).wait()
        _chunk_scores(u_rows, rows, scores_all, off, lane, off, KNBR, 3)
        if c + 4 < KNBR:
            pltpu.async_copy(itab_h.at[ids_of(c + 4)], rows, semr)

    for c in range(KNBR):
        step(c, rbufs[c % 4], rsems[c % 4])

    pltpu.sync_copy(scores_all, inner_o.at[pl.ds(fbase, BPW * KNBR)])

    def drainS(c, carry):
        pltpu.make_async_copy(simf_h.at[pidx_of(0)],
                              sim_all.at[pl.ds(0, BPW)], semS).wait()
        return carry
    lax.fori_loop(0, KNBR, drainS, 0)
    pltpu.sync_copy(sim_all, sim_o.at[pl.ds(fbase, BPW * KNBR)])


def _sc_nbr_call(urows, nbrpos, itab, nbrf, simf):
    f32 = jnp.float32
    i32 = jnp.int32
    out_type = [
        jax.ShapeDtypeStruct((B * KNBR,), f32),
        jax.ShapeDtypeStruct((B * KNBR,), f32),
    ]
    scratch_types = [
        pltpu.VMEM((BPW, D), f32),
        pltpu.VMEM((BPW * KNBR,), i32),
        pltpu.VMEM((BPW * KNBR,), i32),
        pltpu.VMEM((BPW, D), f32),
        pltpu.VMEM((BPW, D), f32),
        pltpu.VMEM((BPW, D), f32),
        pltpu.VMEM((BPW, D), f32),
        pltpu.VMEM((BPW * KNBR,), f32),
        pltpu.VMEM((BPW * KNBR,), f32),
    ] + [pltpu.SemaphoreType.DMA] * 7
    fn = pl.kernel(_sc_nbr_body, out_type=out_type, mesh=_sc_mesh(),
                   scratch_types=scratch_types, compiler_params=_SC_PARAMS)
    return fn(urows, nbrpos, itab, nbrf, simf)


def _mk_norm_body(nstep):
    def _norm_body(x_ref, o_ref, acc_ref):
        step = pl.program_id(0)

        @pl.when(step == 0)
        def _():
            acc_ref[0] = 0.0

        x = x_ref[...]
        acc_ref[0] += jnp.sum(x * x)

        @pl.when(step == nstep - 1)
        def _():
            o_ref[...] = jnp.full((1, 1), 0.5 * acc_ref[0], jnp.float32)
    return _norm_body


def _tc_norm(x, nstep, rows, cols):
    return pl.pallas_call(
        _mk_norm_body(nstep),
        grid=(nstep,),
        in_specs=[pl.BlockSpec((rows, cols), lambda i: (i, 0))],
        out_specs=pl.BlockSpec((1, 1), lambda i: (0, 0)),
        out_shape=jax.ShapeDtypeStruct((1, 1), jnp.float32),
        scratch_shapes=[pltpu.SMEM((1,), jnp.float32)],
    )(x)


def _softplus(x):
    return jnp.maximum(x, 0.0) + jnp.log1p(jnp.exp(-jnp.abs(x)))


def _loss_body(ps_ref, n0_ref, bu_ref, bip_ref, negs_ref, bin_ref, bu2_ref,
               inner_ref, sim_ref, w_ref, o_ref):
    ps = ps_ref[...]
    n0 = n0_ref[...]
    pw = W1 + W2 * bu_ref[...] * bip_ref[...]
    pos_sum = jnp.sum(pw * _softplus(-ps))
    nw = W3 + W4 * bu2_ref[...] * bin_ref[...]
    neg_sum = jnp.sum(nw * _softplus(negs_ref[...]))
    loss = pos_sum + (NEG_WEIGHT / NNEG) * neg_sum
    diff = ps - n0
    sp_beta = jnp.mean(jnp.exp(4.0 * diff))
    g_loss = jnp.sum(jnp.logaddexp(0.0, sp_beta * (-diff))) / sp_beta
    w = w_ref[0, 0]
    loss_l = w * loss + (1.0 - w) * g_loss
    loss_i = jnp.sum(sim_ref[...] * _softplus(-inner_ref[...]))
    o_ref[...] = jnp.full((1, 1), loss_l + LAMBDA_ * loss_i, jnp.float32)


def _tc_loss(ps, n0, bu, bip, negs, binv, inner, sim, w):
    args = (ps.reshape(B // 128, 128), n0.reshape(B // 128, 128),
            bu.reshape(B // 128, 128), bip.reshape(B // 128, 128),
            negs.reshape(B, NNEG), binv.reshape(B, NNEG),
            bu.reshape(B, 1),
            inner.reshape(B * KNBR // 128, 128),
            sim.reshape(B * KNBR // 128, 128), w.reshape(1, 1))
    return pl.pallas_call(
        _loss_body,
        out_shape=jax.ShapeDtypeStruct((1, 1), jnp.float32),
    )(*args)


def kernel(users, pos_items, neg_items, epoch, user_table, item_table,
           weight1, weight2, weight3, beta_uD, beta_iD,
           ii_neighbor_mat, ii_constraint_mat):
    users = users.astype(jnp.int32)
    pos = pos_items.astype(jnp.int32)
    negf = neg_items.reshape(-1).astype(jnp.int32)
    neg0 = neg_items[:, 0].astype(jnp.int32)
    nbrf = ii_neighbor_mat.T.reshape(-1).astype(jnp.int32)
    simf = ii_constraint_mat.T.reshape(-1)
    uflatT = user_table.T.reshape(-1)
    nbrpos = (pos[:, None]
              + jnp.arange(KNBR, dtype=jnp.int32)[None, :] * NITEM
              ).reshape(-1)

    ps, n0, bu, bip, negs, binv, urows = _sc_main_call(
        users, pos, negf, neg0, uflatT, item_table, beta_uD, beta_iD)
    inner, sim = _sc_nbr_call(urows, nbrpos, item_table, nbrf, simf)

    norm = (_tc_norm(user_table.T, 8, D // 8, NUSER)
            + _tc_norm(item_table, 25, NITEM // 25, D))
    w = jnp.minimum(jnp.float32(1.0), jnp.float32(epoch) / 30.0)
    loss_l = _tc_loss(ps, n0, bu, bip, negs, binv, inner, sim, w)

    wsq = 0.5 * (weight1 * weight1 + weight2 * weight2 + weight3 * weight3)
    return loss_l[0, 0] + GAMMA_REG * (norm[0, 0] + wsq)


# speedup vs baseline: 1.0428x; 1.0428x over previous
import jax
import jax.numpy as jnp
from jax import lax
from jax.experimental import pallas as pl
from jax.experimental.pallas import tpu as pltpu
from jax.experimental.pallas import tpu_sc as plsc

D = 64
B = 4096
NNEG = 50
KNBR = 10
NITEM = 100000
NUSER = 100000
W1 = 1e-6
W2 = 1.0
W3 = 1e-6
W4 = 1.0
NEG_WEIGHT = 10.0
GAMMA_REG = 1e-4
LAMBDA_ = 1e-3

NC = 2
NS = 16
L = 16
NW = NC * NS
BPW = B // NW
NV = BPW // L

_SC_PARAMS = pltpu.CompilerParams(use_tc_tiling_on_sc=False)


def _sc_mesh():
    return plsc.VectorSubcoreMesh(
        core_axis_name="c", subcore_axis_name="s",
        num_cores=NC, num_subcores=NS)


def _gather_u_rows(uflatT_h, u_idx, idxT, uT_rows, u_rows, sem, lane):
    def mk_idx(r, carry):
        d = r
        for v in range(NV):
            sl = pl.ds(v * L, L)
            idxT[r, sl] = u_idx[sl] + d * NUSER
        return carry
    lax.fori_loop(0, D, mk_idx, 0)

    def fire(d, carry):
        pltpu.async_copy(uflatT_h.at[idxT.at[d]], uT_rows.at[d], sem)
        return carry
    lax.fori_loop(0, D, fire, 0)

    def drain(d, carry):
        pltpu.make_async_copy(uflatT_h.at[idxT.at[0]], uT_rows.at[0],
                              sem).wait()
        return carry
    lax.fori_loop(0, D, drain, 0)

    masks = [((lane & s) == 0) for s in (8, 4, 2, 1)]

    def tr_block(bj, carry):
        col0 = bj * L
        for bi in range(D // L):
            v = [uT_rows[bi * L + r, pl.ds(col0, L)] for r in range(L)]
            for si, s in enumerate((8, 4, 2, 1)):
                m = masks[si]
                for r in range(L):
                    if r & s == 0:
                        a, b = v[r], v[r | s]
                        v[r] = jnp.where(m, a, b[lane ^ s])
                        v[r | s] = jnp.where(m, a[lane ^ s], b)
            for r in range(L):
                u_rows[col0 + r, pl.ds(bi * L, L)] = v[r]
        return carry
    lax.fori_loop(0, NV, tr_block, 0)


def _chunk_scores(u_rows, rows_ref, score_ref, sbase, lane, flat0, q, nsel):
    NK = D // L

    def dot_with(uslot, j):
        acc = None
        for k in range(NK):
            t = uslot(k) * rows_ref[j, pl.ds(k * L, L)]
            acc = t if acc is None else acc + t
        for sh in (8, 4, 2, 1):
            acc = acc + acc[lane ^ sh]
        return acc

    if q == 1:
        def body(j0, carry):
            vec = jnp.zeros((L,), jnp.float32)
            for jj in range(L):
                j = j0 * L + jj
                vec = jnp.where(
                    lane == jj,
                    dot_with(lambda k: u_rows[j, pl.ds(k * L, L)], j), vec)
            score_ref[pl.ds(sbase + j0 * L, L)] = vec
            return carry
        lax.fori_loop(0, NV, body, 0)
        return

    def body(j0, carry):
        g0 = flat0 + j0 * L
        b0 = g0 // q
        ubufs = []
        for t in range(nsel):
            bt = jnp.minimum(b0 + t, BPW - 1)
            ubufs.append([u_rows[bt, pl.ds(k * L, L)] for k in range(4)])
        vec = jnp.zeros((L,), jnp.float32)
        r0 = g0 - b0 * q
        for jj in range(L):
            s = (r0 + jj) // q

            def uslot(k, _s=s):
                uk = ubufs[0][k]
                for t in range(1, nsel):
                    uk = jnp.where(_s == t, ubufs[t][k], uk)
                return uk
            vec = jnp.where(lane == jj, dot_with(uslot, j0 * L + jj), vec)
        score_ref[pl.ds(sbase + j0 * L, L)] = vec
        return carry
    lax.fori_loop(0, NV, body, 0)


def _sc_main_body(users_h, pos_h, negf_h, neg0_h, uflatT_h, itab_h,
                  bu_h, bi_h,
                  ps_o, n0_o, bu_o, bip_o, negs_o, bin_o,
                  u_idx, p_idx, n0_idx, negf_all, idxT, uT_rows,
                  u_rows, a_rows, b_rows, r0_rows, r1_rows,
                  bu_v, bip_v, scores_all, bin_all, sc_v,
                  semG, semA, semB, semBU, semBIP, semR0, semR1, semF):
    wid = lax.axis_index("s") * NC + lax.axis_index("c")
    base = wid * BPW
    fbase_n = base * NNEG
    lane = lax.iota(jnp.int32, L)

    pltpu.sync_copy(users_h.at[pl.ds(base, BPW)], u_idx)
    pltpu.sync_copy(pos_h.at[pl.ds(base, BPW)], p_idx)
    pltpu.sync_copy(neg0_h.at[pl.ds(base, BPW)], n0_idx)
    cpA = pltpu.async_copy(itab_h.at[p_idx], a_rows, semA)
    cpB = pltpu.async_copy(itab_h.at[n0_idx], b_rows, semB)
    cpBU = pltpu.async_copy(bu_h.at[u_idx], bu_v, semBU)
    cpBIP = pltpu.async_copy(bi_h.at[p_idx], bip_v, semBIP)

    pltpu.sync_copy(negf_h.at[pl.ds(fbase_n, BPW * NNEG)], negf_all)

    def idx_of(c):
        return negf_all.at[pl.ds(c * BPW, BPW)]

    pltpu.async_copy(itab_h.at[idx_of(0)], r0_rows, semR0)
    pltpu.async_copy(itab_h.at[idx_of(1)], r1_rows, semR1)

    _gather_u_rows(uflatT_h, u_idx, idxT, uT_rows, u_rows, semG, lane)

    cpA.wait()
    _chunk_scores(u_rows, a_rows, sc_v, 0, lane, 0, 1, 1)
    pltpu.sync_copy(sc_v, ps_o.at[pl.ds(base, BPW)])
    cpB.wait()
    _chunk_scores(u_rows, b_rows, sc_v, 0, lane, 0, 1, 1)
    pltpu.sync_copy(sc_v, n0_o.at[pl.ds(base, BPW)])
    cpBU.wait()
    pltpu.sync_copy(bu_v, bu_o.at[pl.ds(base, BPW)])
    cpBIP.wait()
    pltpu.sync_copy(bip_v, bip_o.at[pl.ds(base, BPW)])

    def consume(c, rows, semr):
        off = c * BPW
        pltpu.make_async_copy(itab_h.at[idx_of(c)], rows, semr).wait()
        pltpu.async_copy(bi_h.at[idx_of(c)],
                         bin_all.at[pl.ds(off, BPW)], semF)
        _chunk_scores(u_rows, rows, scores_all, off, lane, off, NNEG, 2)

    def outer(cc, carry):
        e = cc * 2
        consume(e, r0_rows, semR0)

        @pl.when(cc < NNEG // 2 - 1)
        def _():
            pltpu.async_copy(itab_h.at[idx_of(e + 2)], r0_rows, semR0)
        consume(e + 1, r1_rows, semR1)

        @pl.when(cc < NNEG // 2 - 1)
        def _():
            pltpu.async_copy(itab_h.at[idx_of(e + 3)], r1_rows, semR1)
        return carry

    lax.fori_loop(0, NNEG // 2, outer, 0)

    pltpu.sync_copy(scores_all, negs_o.at[pl.ds(fbase_n, BPW * NNEG)])

    def drainF(c, carry):
        pltpu.make_async_copy(bi_h.at[idx_of(0)],
                              bin_all.at[pl.ds(0, BPW)], semF).wait()
        return carry
    lax.fori_loop(0, NNEG, drainF, 0)
    pltpu.sync_copy(bin_all, bin_o.at[pl.ds(fbase_n, BPW * NNEG)])


def _sc_main_call(users, pos, negf, neg0, uflatT, itab, bu, bi):
    f32 = jnp.float32
    i32 = jnp.int32
    out_type = [
        jax.ShapeDtypeStruct((B,), f32),
        jax.ShapeDtypeStruct((B,), f32),
        jax.ShapeDtypeStruct((B,), f32),
        jax.ShapeDtypeStruct((B,), f32),
        jax.ShapeDtypeStruct((B * NNEG,), f32),
        jax.ShapeDtypeStruct((B * NNEG,), f32),
    ]
    scratch_types = [
        pltpu.VMEM((BPW,), i32),
        pltpu.VMEM((BPW,), i32),
        pltpu.VMEM((BPW,), i32),
        pltpu.VMEM((BPW * NNEG,), i32),
        pltpu.VMEM((D, BPW), i32),
        pltpu.VMEM((D, BPW), f32),
        pltpu.VMEM((BPW, D), f32),
        pltpu.VMEM((BPW, D), f32),
        pltpu.VMEM((BPW, D), f32),
        pltpu.VMEM((BPW, D), f32),
        pltpu.VMEM((BPW, D), f32),
        pltpu.VMEM((BPW,), f32),
        pltpu.VMEM((BPW,), f32),
        pltpu.VMEM((BPW * NNEG,), f32),
        pltpu.VMEM((BPW * NNEG,), f32),
        pltpu.VMEM((BPW,), f32),
    ] + [pltpu.SemaphoreType.DMA] * 8
    fn = pl.kernel(_sc_main_body, out_type=out_type, mesh=_sc_mesh(),
                   scratch_types=scratch_types, compiler_params=_SC_PARAMS)
    return fn(users, pos, negf, neg0, uflatT, itab, bu, bi)


def _sc_nbr_body(users_h, nbrpos_h, uflatT_h, itab_h, nbrf_h, simf_h,
                 inner_o, sim_o,
                 u_idx, idxT, uT_rows, u_rows, np_all, ids_all,
                 r0_rows, r1_rows, scores_all, sim_all,
                 semG, semI, semR0, semR1, semS):
    wid = lax.axis_index("s") * NC + lax.axis_index("c")
    base = wid * BPW
    fbase = base * KNBR
    lane = lax.iota(jnp.int32, L)

    pltpu.sync_copy(users_h.at[pl.ds(base, BPW)], u_idx)
    pltpu.sync_copy(nbrpos_h.at[pl.ds(fbase, BPW * KNBR)], np_all)

    def pidx_of(c):
        return np_all.at[pl.ds(c * BPW, BPW)]

    def ids_of(c):
        return ids_all.at[pl.ds(c * BPW, BPW)]

    for c in range(KNBR):
        pltpu.async_copy(nbrf_h.at[pidx_of(c)], ids_of(c), semI)
        pltpu.async_copy(simf_h.at[pidx_of(c)],
                         sim_all.at[pl.ds(c * BPW, BPW)], semS)

    _gather_u_rows(uflatT_h, u_idx, idxT, uT_rows, u_rows, semG, lane)

    def drainI(c, carry):
        pltpu.make_async_copy(nbrf_h.at[pidx_of(0)], ids_of(0),
                              semI).wait()
        return carry
    lax.fori_loop(0, KNBR, drainI, 0)

    pltpu.async_copy(itab_h.at[ids_of(0)], r0_rows, semR0)
    pltpu.async_copy(itab_h.at[ids_of(1)], r1_rows, semR1)

    def step(c, rows, semr):
        off = c * BPW
        pltpu.make_async_copy(itab_h.at[ids_of(c)], rows, semr).wait()
        _chunk_scores(u_rows, rows, scores_all, off, lane, off, KNBR, 3)
        if c + 2 < KNBR:
            pltpu.async_copy(itab_h.at[ids_of(c + 2)], rows, semr)

    for c in range(KNBR):
        step(c, (r0_rows, r1_rows)[c % 2], (semR0, semR1)[c % 2])

    pltpu.sync_copy(scores_all, inner_o.at[pl.ds(fbase, BPW * KNBR)])

    def drainS(c, carry):
        pltpu.make_async_copy(simf_h.at[pidx_of(0)],
                              sim_all.at[pl.ds(0, BPW)], semS).wait()
        return carry
    lax.fori_loop(0, KNBR, drainS, 0)
    pltpu.sync_copy(sim_all, sim_o.at[pl.ds(fbase, BPW * KNBR)])


def _sc_nbr_call(users, nbrpos, uflatT, itab, nbrf, simf):
    f32 = jnp.float32
    i32 = jnp.int32
    out_type = [
        jax.ShapeDtypeStruct((B * KNBR,), f32),
        jax.ShapeDtypeStruct((B * KNBR,), f32),
    ]
    scratch_types = [
        pltpu.VMEM((BPW,), i32),
        pltpu.VMEM((D, BPW), i32),
        pltpu.VMEM((D, BPW), f32),
        pltpu.VMEM((BPW, D), f32),
        pltpu.VMEM((BPW * KNBR,), i32),
        pltpu.VMEM((BPW * KNBR,), i32),
        pltpu.VMEM((BPW, D), f32),
        pltpu.VMEM((BPW, D), f32),
        pltpu.VMEM((BPW * KNBR,), f32),
        pltpu.VMEM((BPW * KNBR,), f32),
    ] + [pltpu.SemaphoreType.DMA] * 5
    fn = pl.kernel(_sc_nbr_body, out_type=out_type, mesh=_sc_mesh(),
                   scratch_types=scratch_types, compiler_params=_SC_PARAMS)
    return fn(users, nbrpos, uflatT, itab, nbrf, simf)


def _mk_norm_body(nstep):
    def _norm_body(x_ref, o_ref, acc_ref):
        step = pl.program_id(0)

        @pl.when(step == 0)
        def _():
            acc_ref[0] = 0.0

        x = x_ref[...]
        acc_ref[0] += jnp.sum(x * x)

        @pl.when(step == nstep - 1)
        def _():
            o_ref[...] = jnp.full((1, 1), 0.5 * acc_ref[0], jnp.float32)
    return _norm_body


def _tc_norm(x, nstep, rows, cols):
    return pl.pallas_call(
        _mk_norm_body(nstep),
        grid=(nstep,),
        in_specs=[pl.BlockSpec((rows, cols), lambda i: (i, 0))],
        out_specs=pl.BlockSpec((1, 1), lambda i: (0, 0)),
        out_shape=jax.ShapeDtypeStruct((1, 1), jnp.float32),
        scratch_shapes=[pltpu.SMEM((1,), jnp.float32)],
    )(x)


def _softplus(x):
    return jnp.maximum(x, 0.0) + jnp.log1p(jnp.exp(-jnp.abs(x)))


def _loss_body(ps_ref, n0_ref, bu_ref, bip_ref, negs_ref, bin_ref, bu2_ref,
               inner_ref, sim_ref, w_ref, o_ref):
    ps = ps_ref[...]
    n0 = n0_ref[...]
    pw = W1 + W2 * bu_ref[...] * bip_ref[...]
    pos_sum = jnp.sum(pw * _softplus(-ps))
    nw = W3 + W4 * bu2_ref[...] * bin_ref[...]
    neg_sum = jnp.sum(nw * _softplus(negs_ref[...]))
    loss = pos_sum + (NEG_WEIGHT / NNEG) * neg_sum
    diff = ps - n0
    sp_beta = jnp.mean(jnp.exp(4.0 * diff))
    g_loss = jnp.sum(jnp.logaddexp(0.0, sp_beta * (-diff))) / sp_beta
    w = w_ref[0, 0]
    loss_l = w * loss + (1.0 - w) * g_loss
    loss_i = jnp.sum(sim_ref[...] * _softplus(-inner_ref[...]))
    o_ref[...] = jnp.full((1, 1), loss_l + LAMBDA_ * loss_i, jnp.float32)


def _tc_loss(ps, n0, bu, bip, negs, binv, inner, sim, w):
    args = (ps.reshape(B // 128, 128), n0.reshape(B // 128, 128),
            bu.reshape(B // 128, 128), bip.reshape(B // 128, 128),
            negs.reshape(B, NNEG), binv.reshape(B, NNEG),
            bu.reshape(B, 1),
            inner.reshape(B * KNBR // 128, 128),
            sim.reshape(B * KNBR // 128, 128), w.reshape(1, 1))
    return pl.pallas_call(
        _loss_body,
        out_shape=jax.ShapeDtypeStruct((1, 1), jnp.float32),
    )(*args)


def kernel(users, pos_items, neg_items, epoch, user_table, item_table,
           weight1, weight2, weight3, beta_uD, beta_iD,
           ii_neighbor_mat, ii_constraint_mat):
    users = users.astype(jnp.int32)
    pos = pos_items.astype(jnp.int32)
    negf = neg_items.reshape(-1).astype(jnp.int32)
    neg0 = neg_items[:, 0].astype(jnp.int32)
    nbrf = ii_neighbor_mat.T.reshape(-1).astype(jnp.int32)
    simf = ii_constraint_mat.T.reshape(-1)
    uflatT = user_table.T.reshape(-1)
    nbrpos = (pos[:, None]
              + jnp.arange(KNBR, dtype=jnp.int32)[None, :] * NITEM
              ).reshape(-1)

    ps, n0, bu, bip, negs, binv = _sc_main_call(
        users, pos, negf, neg0, uflatT, item_table, beta_uD, beta_iD)
    inner, sim = _sc_nbr_call(users, nbrpos, uflatT, item_table,
                              nbrf, simf)

    norm = (_tc_norm(user_table.T, 8, D // 8, NUSER)
            + _tc_norm(item_table, 25, NITEM // 25, D))
    w = jnp.minimum(jnp.float32(1.0), jnp.float32(epoch) / 30.0)
    loss_l = _tc_loss(ps, n0, bu, bip, negs, binv, inner, sim, w)

    wsq = 0.5 * (weight1 * weight1 + weight2 * weight2 + weight3 * weight3)
    return loss_l[0, 0] + GAMMA_REG * (norm[0, 0] + wsq)
